# Initial kernel scaffold; baseline (speedup 1.0000x reference)
import jax
import jax.numpy as jnp
from jax.experimental import pallas as pl


def kernel(x, table):
    raise NotImplementedError("write your pallas kernel here")


# speedup vs baseline: 1.2912x; 1.2912x over previous
import functools

import jax
import jax.numpy as jnp
from jax import lax
from jax.experimental import pallas as pl
from jax.experimental.pallas import tpu as pltpu
from jax.experimental.pallas import tpu_sc as plsc

_NC = 2
_NS = 16
_NW = _NC * _NS

_GSZ = 128


@functools.partial(jax.jit, static_argnames=("n_g", "ch"))
def _sc_gather(table, idx3, n_g, ch):
    D = table.shape[1]
    n_ch = n_g // ch
    per_w = n_g * _GSZ
    rows_per_ch = ch * _GSZ
    mesh = plsc.VectorSubcoreMesh(core_axis_name="c", subcore_axis_name="s")

    @functools.partial(
        pl.kernel,
        out_type=jax.ShapeDtypeStruct((_NW * per_w, D), jnp.float32),
        mesh=mesh,
        compiler_params=pltpu.CompilerParams(use_tc_tiling_on_sc=False),
        scratch_types=[
            pltpu.VMEM((n_g, _GSZ), jnp.int32),
            pltpu.VMEM((rows_per_ch, D), jnp.float32),
            pltpu.VMEM((rows_per_ch, D), jnp.float32),
            pltpu.SemaphoreType.DMA,
            pltpu.SemaphoreType.DMA,
            pltpu.SemaphoreType.DMA,
            pltpu.SemaphoreType.DMA,
        ],
    )
    def k(table_hbm, idx_hbm, out_hbm, idx_v, buf0, buf1, g0, g1, o0, o1):
        wid = lax.axis_index("s") * _NC + lax.axis_index("c")
        base = wid * per_w
        pltpu.sync_copy(idx_hbm.at[wid], idx_v)
        bufs = (buf0, buf1)
        gsems = (g0, g1)
        osems = (o0, o1)

        def fire(c, b):
            for j in range(ch):
                pltpu.async_copy(
                    table_hbm.at[idx_v.at[c * ch + j]],
                    bufs[b].at[pl.ds(j * _GSZ, _GSZ)],
                    gsems[b],
                )

        def drain_gathers(b):
            pltpu.make_async_copy(
                out_hbm.at[pl.ds(0, rows_per_ch)], bufs[b], gsems[b]
            ).wait()

        def out_copy(c, b):
            return pltpu.make_async_copy(
                bufs[b],
                out_hbm.at[pl.ds(base + c * rows_per_ch, rows_per_ch)],
                osems[b],
            )

        fire(0, 0)

        @pl.loop(0, n_ch // 2)
        def _(t):
            for b in range(2):
                c = t * 2 + b
                nb = 1 - b
                drain_gathers(b)
                out_copy(c, b).start()

                @pl.when(c >= 1)
                def _():
                    out_copy(c - 1, nb).wait()

                @pl.when(c + 1 < n_ch)
                def _():
                    fire(c + 1, nb)

        out_copy(n_ch - 1, (n_ch - 1) % 2).wait()

    return k(table, idx3)


def kernel(x, table):
    if x.ndim != 4:
        raise ValueError(f"TokenEmbedding expects 4D input [B, H, W, C]. Got: {x.shape}")
    V, D = table.shape
    if x.shape[-1] == V:
        idx = jnp.argmax(x, axis=-1).astype(jnp.int32)
    else:
        idx = x.astype(jnp.int32)
    B, H, W = x.shape[0], x.shape[1], x.shape[2]
    N = B * H * W * (1 if x.shape[-1] == V else x.shape[-1])
    flat = idx.reshape(N)

    assert N % (_NW * _GSZ) == 0, (N,)
    n_g = N // (_NW * _GSZ)
    ch = 1
    for cand in range(2, 9):
        if n_g % cand == 0 and (n_g // cand) % 2 == 0:
            ch = cand
    idx3 = flat.reshape(_NW, n_g, _GSZ)
    out = _sc_gather(table, idx3, n_g, ch)
    return out.reshape(B, H, W, D)
